# Initial kernel scaffold; baseline (speedup 1.0000x reference)
import jax
import jax.numpy as jnp
from jax.experimental import pallas as pl


def kernel(np_emb, rp_emb, W_lan, a_src, a_dst, W_c1, b_c1, W_c2, b_c2, W_fc, b_fc, pairs_id, edges_np, node_id):
    raise NotImplementedError("write your pallas kernel here")


# speedup vs baseline: 9.7021x; 9.7021x over previous
import functools

import jax
import jax.numpy as jnp
from jax import lax
from jax.experimental import pallas as pl
from jax.experimental.pallas import tpu as pltpu
from jax.experimental.pallas import tpu_sc as plsc

N_NODES = 10000
N_PAD = 10240
D = 128
N_EDGES = 320000
BS = 1024
CC = 32
R2 = 16

NC = 2
NS = 16
NW = NC * NS
EPT = N_EDGES // NW
CH = 80
NCHUNK = EPT // CH
SB = 25
NSB = NCHUNK // SB
SROW = N_PAD // 16
BT = 64
NBLK = BS // BT
SN = 1280


def _proj_body(x_ref, w_ref, asd_ref, h_ref, hsd_ref):
    h = jnp.dot(x_ref[...], w_ref[...], preferred_element_type=jnp.float32)
    h_ref[...] = h
    hsd_ref[...] = jnp.dot(h, asd_ref[...], preferred_element_type=jnp.float32)


_proj = pl.pallas_call(
    _proj_body,
    grid=(N_PAD // 128,),
    in_specs=[
        pl.BlockSpec((128, D), lambda i: (i, 0)),
        pl.BlockSpec((D, D), lambda i: (0, 0)),
        pl.BlockSpec((D, 2), lambda i: (0, 0)),
    ],
    out_specs=[
        pl.BlockSpec((128, D), lambda i: (i, 0)),
        pl.BlockSpec((128, 2), lambda i: (i, 0)),
    ],
    out_shape=[
        jax.ShapeDtypeStruct((N_PAD, D), jnp.float32),
        jax.ShapeDtypeStruct((N_PAD, 2), jnp.float32),
    ],
)


_sc_mesh = plsc.VectorSubcoreMesh(core_axis_name="c", subcore_axis_name="s")


@functools.partial(
    pl.kernel,
    out_type=(
        jax.ShapeDtypeStruct((NC, NS, 64, D), jnp.float32),
        jax.ShapeDtypeStruct((NC, NS, 64), jnp.float32),
        jax.ShapeDtypeStruct((BS, D), jnp.float32),
        jax.ShapeDtypeStruct((BS, D), jnp.float32),
    ),
    mesh=_sc_mesh,
    compiler_params=pltpu.CompilerParams(needs_layout_passes=False,
                                         use_tc_tiling_on_sc=False),
    scratch_types=[
        pltpu.VMEM((SB, CH), jnp.int32),
        pltpu.VMEM((SB, CH), jnp.int32),
        pltpu.VMEM((SROW, 16), jnp.float32),
        pltpu.VMEM((SROW, 16), jnp.float32),
        pltpu.VMEM((SROW, 16), jnp.float32),
        pltpu.VMEM((CH + 16,), jnp.float32),
        pltpu.VMEM((CH, D), jnp.float32),
        pltpu.VMEM((5, 128), jnp.int32),
        pltpu.VMEM((64,), jnp.int32),
        pltpu.VMEM((64,), jnp.float32),
        pltpu.VMEM_SHARED((N_NODES, D), jnp.float32),
        pltpu.VMEM_SHARED((SROW, 16), jnp.float32),
        pltpu.SemaphoreType.DMA,
    ],
)
def _sc_edge(src3_h, dst3_h, hs_h, hd_h, h_h, x_h, rp_h, p0_h, p1_h,
             ridx_h, zrow_h, zs_h,
             up_o, sp_o, sub_o, rel_o,
             src2_v, dst2_v, hs_v, hd_v, spart, exb, rowbuf, ridx_v,
             pv, sg, U_sh, s_sh, sem):
    c = lax.axis_index("c")
    s = lax.axis_index("s")
    wid = c * NS + s

    pltpu.sync_copy(hs_h, hs_v)
    pltpu.sync_copy(hd_h, hd_v)
    pltpu.sync_copy(ridx_h, ridx_v)
    pltpu.sync_copy(zs_h, spart)
    pltpu.sync_copy(zrow_h, U_sh.at[pl.ds(s * 625, 625)])
    pltpu.sync_copy(zs_h.at[pl.ds(s * 40, 40)], s_sh.at[pl.ds(s * 40, 40)])
    plsc.subcore_barrier()

    def chunk(ci, carry):
        for k in range(5):
            off = k * 16
            sv = src2_v[ci, pl.ds(off, 16)]
            dv = dst2_v[ci, pl.ds(off, 16)]
            sh = lax.shift_right_logical(sv, 4)
            sl = lax.bitwise_and(sv, 15)
            dh = lax.shift_right_logical(dv, 4)
            dl = lax.bitwise_and(dv, 15)
            hsg = plsc.load_gather(hs_v, [sh, sl])
            hdg = plsc.load_gather(hd_v, [dh, dl])
            v = hsg + hdg
            ex = jnp.exp(jnp.where(v >= 0.0, v, 0.2 * v))
            exb[pl.ds(k * 16, 16)] = ex
            plsc.addupdate_scatter(spart, [dh, dl], ex)
        pltpu.async_copy(h_h.at[src2_v.at[ci]], rowbuf, sem).wait()

        def rowscale(r, cr):
            w = exb[pl.ds(r, 16)][0]
            for cc2 in range(8):
                sl2 = pl.ds(cc2 * 16, 16)
                rowbuf[r, sl2] = rowbuf[r, sl2] * w
            return cr

        lax.fori_loop(0, CH, rowscale, 0, unroll=4)
        pltpu.sync_copy(rowbuf, U_sh.at[dst2_v.at[ci]], add=True)
        return carry

    def superblock(si, carry):
        pltpu.sync_copy(src3_h.at[wid, pl.ds(si * SB, SB)], src2_v)
        pltpu.sync_copy(dst3_h.at[wid, pl.ds(si * SB, SB)], dst2_v)
        lax.fori_loop(0, SB, chunk, 0)
        return carry

    lax.fori_loop(0, NSB, superblock, 0)

    for j in range(5):
        pltpu.sync_copy(spart.at[pl.ds(j * 128, 128)], s_sh.at[ridx_v.at[j]],
                        add=True)
    plsc.subcore_barrier()

    base = s * 64
    gbuf = rowbuf.at[pl.ds(0, 64)]
    pltpu.sync_copy(p0_h.at[pl.ds(base, 64)], pv)
    pltpu.async_copy(U_sh.at[pv], gbuf, sem).wait()
    pltpu.sync_copy(gbuf, up_o.at[c, s])
    pltpu.sync_copy(s_sh, spart)
    for g in range(4):
        pidx = pv[pl.ds(g * 16, 16)]
        sg[pl.ds(g * 16, 16)] = plsc.load_gather(
            spart,
            [lax.shift_right_logical(pidx, 4), lax.bitwise_and(pidx, 15)])
    pltpu.sync_copy(sg, sp_o.at[c, s])

    @pl.when(c == 0)
    def _():
        pltpu.async_copy(x_h.at[pv], gbuf, sem).wait()
        pltpu.sync_copy(gbuf, sub_o.at[pl.ds(base, 64)])

    @pl.when(c == 1)
    def _():
        pltpu.sync_copy(p1_h.at[pl.ds(base, 64)], pv)
        pltpu.async_copy(rp_h.at[pv], gbuf, sem).wait()
        pltpu.sync_copy(gbuf, rel_o.at[pl.ds(base, 64)])


M = BT * D


def _conv_body(sub_ref, rel_ref, u_ref, srep_ref, wc1_ref, bc1_ref, a_ref,
               bc2_ref, pr_ref, pool_ref):
    canon = (u_ref[0:1, :] + u_ref[1:2, :]) / (srep_ref[...] + 1e-9)
    x = jnp.concatenate([sub_ref[...], rel_ref[...], canon], axis=0)
    y = jnp.dot(wc1_ref[...], x, preferred_element_type=jnp.float32)
    y = jax.nn.relu(y + bc1_ref[...])
    col = lax.broadcasted_iota(jnp.int32, (R2, M), 1)
    wpos = lax.bitwise_and(col, D - 1)
    zero = jnp.zeros((R2, 1), jnp.float32)
    ym = jnp.where(wpos == 0, 0.0,
                   jnp.concatenate([zero, y[:, :-1]], axis=1))
    yp = jnp.where(wpos == D - 1, 0.0,
                   jnp.concatenate([y[:, 1:], zero], axis=1))
    z = jnp.dot(a_ref[0], ym, preferred_element_type=jnp.float32)
    z = z + jnp.dot(a_ref[1], y, preferred_element_type=jnp.float32)
    z = z + jnp.dot(a_ref[2], yp, preferred_element_type=jnp.float32)
    z = jax.nn.relu(z + bc2_ref[...])
    zh = z.reshape(128, 4, M).sum(axis=1)
    zf = zh.reshape(128, BT, D).reshape(128 * BT, D)
    q = jnp.dot(zf, pr_ref[...], preferred_element_type=jnp.float32)
    pool_ref[...] = q.reshape(128, BT, CC)


_conv = pl.pallas_call(
    _conv_body,
    grid=(NBLK,),
    in_specs=[
        pl.BlockSpec((1, M), lambda i: (0, i)),
        pl.BlockSpec((1, M), lambda i: (0, i)),
        pl.BlockSpec((NC, M), lambda i: (0, i)),
        pl.BlockSpec((1, M), lambda i: (0, i)),
        pl.BlockSpec((R2, 3), lambda i: (0, 0)),
        pl.BlockSpec((R2, 1), lambda i: (0, 0)),
        pl.BlockSpec((3, 512, R2), lambda i: (0, 0, 0)),
        pl.BlockSpec((512, 1), lambda i: (0, 0)),
        pl.BlockSpec((D, CC), lambda i: (0, 0)),
    ],
    out_specs=pl.BlockSpec((128, BT, CC), lambda i: (0, i, 0)),
    out_shape=jax.ShapeDtypeStruct((128, BS, CC), jnp.float32),
)


def _fc_body(t_ref, wfc_ref, bfc_ref, obj_ref):
    obj = jnp.dot(t_ref[...], wfc_ref[...], preferred_element_type=jnp.float32)
    obj_ref[...] = jax.nn.relu(obj + bfc_ref[...])


_fc = pl.pallas_call(
    _fc_body,
    out_shape=jax.ShapeDtypeStruct((BS, D), jnp.float32),
)


def _score_body(obj_ref, emb_ref, out_ref):
    out_ref[...] = lax.dot_general(
        obj_ref[...], emb_ref[...], (((1,), (1,)), ((), ())),
        preferred_element_type=jnp.float32)


_score = pl.pallas_call(
    _score_body,
    grid=(N_PAD // SN,),
    in_specs=[
        pl.BlockSpec((BS, D), lambda i: (0, 0)),
        pl.BlockSpec((SN, D), lambda i: (i, 0)),
    ],
    out_specs=pl.BlockSpec((BS, SN), lambda i: (0, i)),
    out_shape=jax.ShapeDtypeStruct((BS, N_PAD), jnp.float32),
)


def kernel(np_emb, rp_emb, W_lan, a_src, a_dst, W_c1, b_c1, W_c2, b_c2,
           W_fc, b_fc, pairs_id, edges_np, node_id):
    f32 = jnp.float32
    x = jnp.take(np_emb.astype(f32), node_id, axis=0)
    x_pad = jnp.pad(x, ((0, N_PAD - N_NODES), (0, 0)))
    asd = jnp.stack([a_src.astype(f32), a_dst.astype(f32)], axis=1)

    h_pad, hsd = _proj(x_pad, W_lan.astype(f32), asd)
    hs = hsd[:, 0].reshape(SROW, 16)
    hd = hsd[:, 1].reshape(SROW, 16)

    src3 = edges_np[0].astype(jnp.int32).reshape(NW, NCHUNK, CH)
    dst3 = edges_np[1].astype(jnp.int32).reshape(NW, NCHUNK, CH)
    p0 = pairs_id[:, 0].astype(jnp.int32)
    p1 = pairs_id[:, 1].astype(jnp.int32)
    ridx = jnp.arange(SROW, dtype=jnp.int32).reshape(5, 128)
    zrow = jnp.zeros((625, D), f32)
    zs = jnp.zeros((SROW, 16), f32)

    up, sp, sub, rel = _sc_edge(src3, dst3, hs, hd, h_pad, x_pad,
                                rp_emb.astype(f32), p0, p1, ridx, zrow, zs)
    u_f = up.reshape(NC, BS * D)
    s_sum = sp.reshape(NC, BS).sum(axis=0)
    srep = jnp.repeat(s_sum, D).reshape(1, BS * D)
    sub_f = sub.reshape(1, BS * D)
    rel_f = rel.reshape(1, BS * D)

    w2 = W_c2.astype(f32).reshape(CC, 3, 3)
    o_idx = jnp.arange(512) // R2
    h_idx = jnp.arange(512) % R2
    dlt = jnp.arange(R2)[None, :] - h_idx[:, None] + 1
    valid = (dlt >= 0) & (dlt <= 2)
    afull = w2[o_idx[:, None], jnp.clip(dlt, 0, 2), :]
    afull = jnp.where(valid[:, :, None], afull, 0.0)
    amat = jnp.moveaxis(afull, 2, 0)
    bc2rep = jnp.repeat(b_c2.astype(f32), R2).reshape(512, 1)
    prmat = ((jnp.arange(D)[:, None] // 4 == jnp.arange(CC)[None, :])
             .astype(f32) / 16.0)

    pooled = _conv(sub_f, rel_f, u_f, srep, W_c1.astype(f32),
                   b_c1.astype(f32).reshape(R2, 1), amat, bc2rep, prmat)
    t = pooled.transpose(1, 0, 2).reshape(BS, 4096)
    obj = _fc(t, W_fc.astype(f32), b_fc.astype(f32).reshape(1, D))

    scores = _score(obj, x_pad)
    return scores[:, :N_NODES]


# speedup vs baseline: 10.9048x; 1.1240x over previous
import functools

import jax
import jax.numpy as jnp
from jax import lax
from jax.experimental import pallas as pl
from jax.experimental.pallas import tpu as pltpu
from jax.experimental.pallas import tpu_sc as plsc

N_NODES = 10000
N_PAD = 10240
D = 128
N_EDGES = 320000
BS = 1024
CC = 32
R2 = 16

NC = 2
NS = 16
NW = NC * NS
EPT = N_EDGES // NW
CH = 80
NCHUNK = EPT // CH
SB = 25
NSB = NCHUNK // SB
SROW = N_PAD // 16
BT = 64
NBLK = BS // BT
SN = 1280


def _proj_body(x_ref, w_ref, asd_ref, h_ref, hsd_ref):
    h = jnp.dot(x_ref[...], w_ref[...], preferred_element_type=jnp.float32)
    h_ref[...] = h
    hsd_ref[...] = jnp.dot(h, asd_ref[...], preferred_element_type=jnp.float32)


_proj = pl.pallas_call(
    _proj_body,
    grid=(N_PAD // 128,),
    in_specs=[
        pl.BlockSpec((128, D), lambda i: (i, 0)),
        pl.BlockSpec((D, D), lambda i: (0, 0)),
        pl.BlockSpec((D, 2), lambda i: (0, 0)),
    ],
    out_specs=[
        pl.BlockSpec((128, D), lambda i: (i, 0)),
        pl.BlockSpec((128, 2), lambda i: (i, 0)),
    ],
    out_shape=[
        jax.ShapeDtypeStruct((N_PAD, D), jnp.float32),
        jax.ShapeDtypeStruct((N_PAD, 2), jnp.float32),
    ],
)


_sc_mesh = plsc.VectorSubcoreMesh(core_axis_name="c", subcore_axis_name="s")


@functools.partial(
    pl.kernel,
    out_type=(
        jax.ShapeDtypeStruct((NC, NS, 64, D), jnp.float32),
        jax.ShapeDtypeStruct((NC, NS, 64), jnp.float32),
        jax.ShapeDtypeStruct((BS, D), jnp.float32),
        jax.ShapeDtypeStruct((BS, D), jnp.float32),
    ),
    mesh=_sc_mesh,
    compiler_params=pltpu.CompilerParams(needs_layout_passes=False,
                                         use_tc_tiling_on_sc=False),
    scratch_types=[
        pltpu.VMEM((SB, CH), jnp.int32),
        pltpu.VMEM((SB, CH), jnp.int32),
        pltpu.VMEM((SROW, 16), jnp.float32),
        pltpu.VMEM((SROW, 16), jnp.float32),
        pltpu.VMEM((SROW, 16), jnp.float32),
        pltpu.VMEM((CH + 16,), jnp.float32),
        pltpu.VMEM((CH, D), jnp.float32),
        pltpu.VMEM((5, 128), jnp.int32),
        pltpu.VMEM((64,), jnp.int32),
        pltpu.VMEM((64,), jnp.float32),
        pltpu.VMEM_SHARED((N_NODES, D), jnp.float32),
        pltpu.VMEM_SHARED((SROW, 16), jnp.float32),
        pltpu.SemaphoreType.DMA,
    ],
)
def _sc_edge(src3_h, dst3_h, hs_h, hd_h, h_h, x_h, rp_h, p0_h, p1_h,
             ridx_h, zrow_h, zs_h,
             up_o, sp_o, sub_o, rel_o,
             src2_v, dst2_v, hs_v, hd_v, spart, exb, rowbuf, ridx_v,
             pv, sg, U_sh, s_sh, sem):
    c = lax.axis_index("c")
    s = lax.axis_index("s")
    wid = c * NS + s

    pltpu.sync_copy(hs_h, hs_v)
    pltpu.sync_copy(hd_h, hd_v)
    pltpu.sync_copy(ridx_h, ridx_v)
    pltpu.sync_copy(zs_h, spart)
    pltpu.sync_copy(zrow_h, U_sh.at[pl.ds(s * 625, 625)])
    pltpu.sync_copy(zs_h.at[pl.ds(s * 40, 40)], s_sh.at[pl.ds(s * 40, 40)])
    plsc.subcore_barrier()

    def chunk(ci, carry):
        for k in range(5):
            off = k * 16
            sv = src2_v[ci, pl.ds(off, 16)]
            dv = dst2_v[ci, pl.ds(off, 16)]
            sh = lax.shift_right_logical(sv, 4)
            sl = lax.bitwise_and(sv, 15)
            dh = lax.shift_right_logical(dv, 4)
            dl = lax.bitwise_and(dv, 15)
            hsg = plsc.load_gather(hs_v, [sh, sl])
            hdg = plsc.load_gather(hd_v, [dh, dl])
            v = hsg + hdg
            ex = jnp.exp(jnp.where(v >= 0.0, v, 0.2 * v))
            exb[pl.ds(k * 16, 16)] = ex
            plsc.addupdate_scatter(spart, [dh, dl], ex)
        pltpu.async_copy(h_h.at[src2_v.at[ci]], rowbuf, sem).wait()

        def rowscale(r, cr):
            w = exb[pl.ds(r, 16)][0]
            for cc2 in range(8):
                sl2 = pl.ds(cc2 * 16, 16)
                rowbuf[r, sl2] = rowbuf[r, sl2] * w
            return cr

        lax.fori_loop(0, CH, rowscale, 0, unroll=4)
        pltpu.sync_copy(rowbuf, U_sh.at[dst2_v.at[ci]], add=True)
        return carry

    def superblock(si, carry):
        pltpu.sync_copy(src3_h.at[wid, pl.ds(si * SB, SB)], src2_v)
        pltpu.sync_copy(dst3_h.at[wid, pl.ds(si * SB, SB)], dst2_v)
        lax.fori_loop(0, SB, chunk, 0)
        return carry

    lax.fori_loop(0, NSB, superblock, 0)

    for j in range(5):
        pltpu.sync_copy(spart.at[pl.ds(j * 128, 128)], s_sh.at[ridx_v.at[j]],
                        add=True)
    plsc.subcore_barrier()

    base = s * 64
    gbuf = rowbuf.at[pl.ds(0, 64)]
    pltpu.sync_copy(p0_h.at[pl.ds(base, 64)], pv)
    pltpu.async_copy(U_sh.at[pv], gbuf, sem).wait()
    pltpu.sync_copy(gbuf, up_o.at[c, s])
    pltpu.sync_copy(s_sh, spart)
    for g in range(4):
        pidx = pv[pl.ds(g * 16, 16)]
        sg[pl.ds(g * 16, 16)] = plsc.load_gather(
            spart,
            [lax.shift_right_logical(pidx, 4), lax.bitwise_and(pidx, 15)])
    pltpu.sync_copy(sg, sp_o.at[c, s])

    @pl.when(c == 0)
    def _():
        pltpu.async_copy(x_h.at[pv], gbuf, sem).wait()
        pltpu.sync_copy(gbuf, sub_o.at[pl.ds(base, 64)])

    @pl.when(c == 1)
    def _():
        pltpu.sync_copy(p1_h.at[pl.ds(base, 64)], pv)
        pltpu.async_copy(rp_h.at[pv], gbuf, sem).wait()
        pltpu.sync_copy(gbuf, rel_o.at[pl.ds(base, 64)])


M = BT * D


def _conv_body(sub_ref, rel_ref, u_ref, srep_ref, wc1_ref, bc1_ref, a_ref,
               bc2_ref, pr_ref, pool_ref):
    canon = (u_ref[0:1, :] + u_ref[1:2, :]) / (srep_ref[...] + 1e-9)
    x = jnp.concatenate([sub_ref[...], rel_ref[...], canon], axis=0)
    y = jnp.dot(wc1_ref[...], x, preferred_element_type=jnp.float32)
    y = jax.nn.relu(y + bc1_ref[...])
    col = lax.broadcasted_iota(jnp.int32, (R2, M), 1)
    wpos = lax.bitwise_and(col, D - 1)
    zero = jnp.zeros((R2, 1), jnp.float32)
    ym = jnp.where(wpos == 0, 0.0,
                   jnp.concatenate([zero, y[:, :-1]], axis=1))
    yp = jnp.where(wpos == D - 1, 0.0,
                   jnp.concatenate([y[:, 1:], zero], axis=1))
    g = jnp.concatenate([ym, y, yp], axis=0)
    z = jnp.dot(a_ref[...], g, preferred_element_type=jnp.float32)
    z = jax.nn.relu(z + bc2_ref[...])
    zh = z.reshape(128, 4, M).sum(axis=1)
    zf = zh.reshape(128, BT, D).reshape(128 * BT, D)
    q = jnp.dot(zf, pr_ref[...], preferred_element_type=jnp.float32)
    pool_ref[...] = q.reshape(128, BT, CC)


_conv = pl.pallas_call(
    _conv_body,
    grid=(NBLK,),
    in_specs=[
        pl.BlockSpec((1, M), lambda i: (0, i)),
        pl.BlockSpec((1, M), lambda i: (0, i)),
        pl.BlockSpec((NC, M), lambda i: (0, i)),
        pl.BlockSpec((1, M), lambda i: (0, i)),
        pl.BlockSpec((R2, 3), lambda i: (0, 0)),
        pl.BlockSpec((R2, 1), lambda i: (0, 0)),
        pl.BlockSpec((512, 3 * R2), lambda i: (0, 0)),
        pl.BlockSpec((512, 1), lambda i: (0, 0)),
        pl.BlockSpec((D, CC), lambda i: (0, 0)),
    ],
    out_specs=pl.BlockSpec((128, BT, CC), lambda i: (0, i, 0)),
    out_shape=jax.ShapeDtypeStruct((128, BS, CC), jnp.float32),
)


def _fc_body(p_ref, wfc_ref, bfc_ref, obj_ref):
    acc = bfc_ref[...]
    for k in range(128):
        acc = acc + jnp.dot(p_ref[k], wfc_ref[pl.ds(k * CC, CC), :],
                            preferred_element_type=jnp.float32)
    obj_ref[...] = jax.nn.relu(acc)


_fc = pl.pallas_call(
    _fc_body,
    grid=(4,),
    in_specs=[
        pl.BlockSpec((128, BS // 4, CC), lambda i: (0, i, 0)),
        pl.BlockSpec((4096, D), lambda i: (0, 0)),
        pl.BlockSpec((1, D), lambda i: (0, 0)),
    ],
    out_specs=pl.BlockSpec((BS // 4, D), lambda i: (i, 0)),
    out_shape=jax.ShapeDtypeStruct((BS, D), jnp.float32),
)


def _score_body(obj_ref, emb_ref, out_ref):
    out_ref[...] = lax.dot_general(
        obj_ref[...], emb_ref[...], (((1,), (1,)), ((), ())),
        preferred_element_type=jnp.float32)


_score = pl.pallas_call(
    _score_body,
    grid=(N_PAD // SN,),
    in_specs=[
        pl.BlockSpec((BS, D), lambda i: (0, 0)),
        pl.BlockSpec((SN, D), lambda i: (i, 0)),
    ],
    out_specs=pl.BlockSpec((BS, SN), lambda i: (0, i)),
    out_shape=jax.ShapeDtypeStruct((BS, N_PAD), jnp.float32),
)


def kernel(np_emb, rp_emb, W_lan, a_src, a_dst, W_c1, b_c1, W_c2, b_c2,
           W_fc, b_fc, pairs_id, edges_np, node_id):
    f32 = jnp.float32
    x = jnp.take(np_emb.astype(f32), node_id, axis=0)
    x_pad = jnp.pad(x, ((0, N_PAD - N_NODES), (0, 0)))
    asd = jnp.stack([a_src.astype(f32), a_dst.astype(f32)], axis=1)

    h_pad, hsd = _proj(x_pad, W_lan.astype(f32), asd)
    hs = hsd[:, 0].reshape(SROW, 16)
    hd = hsd[:, 1].reshape(SROW, 16)

    src3 = edges_np[0].astype(jnp.int32).reshape(NW, NCHUNK, CH)
    dst3 = edges_np[1].astype(jnp.int32).reshape(NW, NCHUNK, CH)
    p0 = pairs_id[:, 0].astype(jnp.int32)
    p1 = pairs_id[:, 1].astype(jnp.int32)
    ridx = jnp.arange(SROW, dtype=jnp.int32).reshape(5, 128)
    zrow = jnp.zeros((625, D), f32)
    zs = jnp.zeros((SROW, 16), f32)

    up, sp, sub, rel = _sc_edge(src3, dst3, hs, hd, h_pad, x_pad,
                                rp_emb.astype(f32), p0, p1, ridx, zrow, zs)
    u_f = up.reshape(NC, BS * D)
    s_sum = sp.reshape(NC, BS).sum(axis=0)
    srep = jnp.repeat(s_sum, D).reshape(1, BS * D)
    sub_f = sub.reshape(1, BS * D)
    rel_f = rel.reshape(1, BS * D)

    w2 = W_c2.astype(f32).reshape(CC, 3, 3)
    o_idx = jnp.arange(512) // R2
    h_idx = jnp.arange(512) % R2
    dlt = jnp.arange(R2)[None, :] - h_idx[:, None] + 1
    valid = (dlt >= 0) & (dlt <= 2)
    afull = w2[o_idx[:, None], jnp.clip(dlt, 0, 2), :]
    afull = jnp.where(valid[:, :, None], afull, 0.0)
    amat = jnp.moveaxis(afull, 2, 0)
    amat = jnp.concatenate([amat[0], amat[1], amat[2]], axis=1)
    bc2rep = jnp.repeat(b_c2.astype(f32), R2).reshape(512, 1)
    prmat = ((jnp.arange(D)[:, None] // 4 == jnp.arange(CC)[None, :])
             .astype(f32) / 16.0)

    pooled = _conv(sub_f, rel_f, u_f, srep, W_c1.astype(f32),
                   b_c1.astype(f32).reshape(R2, 1), amat, bc2rep, prmat)
    obj = _fc(pooled, W_fc.astype(f32), b_fc.astype(f32).reshape(1, D))

    scores = _score(obj, x_pad)
    return scores[:, :N_NODES]
